# baseline (speedup 1.0000x reference)
import functools

import jax
import jax.numpy as jnp
from jax.experimental import pallas as pl
from jax.experimental.pallas import tpu as pltpu

N = 512
H = 4
C = 64
F_IN = 128
D = H * C
IBLK = 64
G = 8
NEG_SLOPE = 0.2


def _layernorm(x, g, b, eps=1e-5):
    m = jnp.mean(x, axis=-1, keepdims=True)
    v = jnp.mean((x - m) ** 2, axis=-1, keepdims=True)
    return (x - m) / jnp.sqrt(v + eps) * g + b


def _gat_layer(xin, Wl_ref, bl_ref, Wr_ref, br_ref, att_ref, absatt_ref,
               sgn_ref, sgn8_ref, bias_ref, out_ref, xlh_s, xrh_s, xlt_s):
    xl = jnp.dot(xin, Wl_ref[:, :], preferred_element_type=jnp.float32) + bl_ref[0, :]
    xr = jnp.dot(xin, Wr_ref[:, :], preferred_element_type=jnp.float32) + br_ref[0, :]
    vl = xl * absatt_ref[0, :]
    vr = xr * absatt_ref[0, :]

    for h in range(H):
        xlh_s[h] = xl[:, h * C:(h + 1) * C]
        xrh_s[h] = vr[:, h * C:(h + 1) * C].astype(jnp.bfloat16)
        xlt_s[h] = vl[:, h * C:(h + 1) * C].T.astype(jnp.bfloat16)

    for h in range(H):
        xl_h = xlh_s[h]
        vl_hT = xlt_s[h]
        sgn_col = jnp.reshape(sgn_ref[h, :], (C, 1))
        al_row = jnp.sum(vl_hT.astype(jnp.float32) * sgn_col, axis=0,
                         keepdims=True)
        base = 0.6 * al_row
        sgn8_h = sgn8_ref[h].astype(jnp.bfloat16)

        def body(j, _, xl_h=xl_h, vl_hT=vl_hT, base=base, sgn8_h=sgn8_h, h=h):
            i0 = j * IBLK
            vr_blk = xrh_s[h, pl.ds(i0, IBLK), :]
            v = vr_blk[:, :, None] + vl_hT[None, :, :]
            absv = jnp.abs(v).reshape(IBLK // G, G * C, N)
            parts = [
                jax.lax.dot_general(
                    sgn8_h, absv[g], (((1,), (0,)), ((), ())),
                    preferred_element_type=jnp.float32)
                for g in range(IBLK // G)
            ]
            e = 0.4 * jnp.concatenate(parts, axis=0) + base
            m = jnp.max(e, axis=1, keepdims=True)
            p = jnp.exp(e - m)
            den = jnp.sum(p, axis=1, keepdims=True)
            o = jnp.dot(p, xl_h, preferred_element_type=jnp.float32) / den
            out_ref[pl.ds(i0, IBLK), h * C:(h + 1) * C] = o
            return 0

        jax.lax.fori_loop(0, N // IBLK, body, 0)

    return out_ref[:, :] + bias_ref[0, :]


def _kernel_body(x_ref, Wl1_ref, bl1_ref, Wr1_ref, br1_ref, att1_ref, aa1_ref,
                 sg1_ref, sg81_ref, bias1_ref, g1_ref, be1_ref, Wl2_ref, bl2_ref,
                 Wr2_ref, br2_ref, att2_ref, aa2_ref, sg2_ref, sg82_ref,
                 bias2_ref, g2_ref, be2_ref,
                 out_ref, hbuf, xlh_s, xrh_s, xlt_s):
    xb = x_ref[0]
    h1 = _gat_layer(xb, Wl1_ref, bl1_ref, Wr1_ref, br1_ref, att1_ref, aa1_ref,
                    sg1_ref, sg81_ref, bias1_ref, hbuf, xlh_s, xrh_s, xlt_s)
    h1 = jax.nn.relu(_layernorm(h1, g1_ref[0, :], be1_ref[0, :]))
    o = _gat_layer(h1, Wl2_ref, bl2_ref, Wr2_ref, br2_ref, att2_ref, aa2_ref,
                   sg2_ref, sg82_ref, bias2_ref, hbuf, xlh_s, xrh_s, xlt_s)
    out_ref[0] = _layernorm(o, g2_ref[0, :], be2_ref[0, :])


def kernel(x, embedding, Wl1, bl1, Wr1, br1, att1, bias1, g1, be1,
           Wl2, bl2, Wr2, br2, att2, bias2, g2, be2):
    del embedding
    B = x.shape[0]

    row = lambda v: v.reshape(1, -1)
    whole = lambda shape: pl.BlockSpec(shape, lambda b: (0,) * len(shape))

    in_specs = [
            pl.BlockSpec((1, N, F_IN), lambda b: (b, 0, 0)),
            whole((F_IN, D)), whole((1, D)), whole((F_IN, D)), whole((1, D)),
            whole((H, C)), whole((1, D)), whole((H, C)), whole((H, G, G * C)),
            whole((1, D)), whole((1, D)), whole((1, D)),
            whole((D, D)), whole((1, D)), whole((D, D)), whole((1, D)),
            whole((H, C)), whole((1, D)), whole((H, C)), whole((H, G, G * C)),
            whole((1, D)), whole((1, D)), whole((1, D)),
    ]

    call = pl.pallas_call(
        _kernel_body,
        grid=(B,),
        in_specs=in_specs,
        out_specs=pl.BlockSpec((1, N, D), lambda b: (b, 0, 0)),
        out_shape=jax.ShapeDtypeStruct((B, N, D), jnp.float32),
        scratch_shapes=[pltpu.VMEM((N, D), jnp.float32),
                        pltpu.VMEM((H, N, C), jnp.float32),
                        pltpu.VMEM((H, N, C), jnp.bfloat16),
                        pltpu.VMEM((H, C, N), jnp.bfloat16)],
    )
    blockdiag = jax.vmap(lambda s: jnp.kron(jnp.eye(G, dtype=jnp.float32),
                                            s[None, :]))
    sg1 = jnp.sign(att1)
    sg2 = jnp.sign(att2)
    return call(x, Wl1, row(bl1), Wr1, row(br1), att1, row(jnp.abs(att1)),
                sg1, blockdiag(sg1), row(bias1), row(g1), row(be1),
                Wl2, row(bl2), Wr2, row(br2), att2, row(jnp.abs(att2)),
                sg2, blockdiag(sg2), row(bias2), row(g2), row(be2))


# speedup vs baseline: 1.2474x; 1.2474x over previous
import functools

import jax
import jax.numpy as jnp
from jax.experimental import pallas as pl
from jax.experimental.pallas import tpu as pltpu

N = 512
H = 4
C = 64
F_IN = 128
D = H * C
IBLK = 128
G = 8
NEG_SLOPE = 0.2


def _layernorm(x, g, b, eps=1e-5):
    m = jnp.mean(x, axis=-1, keepdims=True)
    v = jnp.mean((x - m) ** 2, axis=-1, keepdims=True)
    return (x - m) / jnp.sqrt(v + eps) * g + b


def _gat_layer(xin, Wl_ref, bl_ref, Wr_ref, br_ref, att_ref, absatt_ref,
               sgn_ref, sgn8_ref, bias_ref, out_ref, xlh_s, xrh_s, xlt_s):
    xl = jnp.dot(xin, Wl_ref[:, :], preferred_element_type=jnp.float32) + bl_ref[0, :]
    xr = jnp.dot(xin, Wr_ref[:, :], preferred_element_type=jnp.float32) + br_ref[0, :]
    vl = xl * absatt_ref[0, :]
    vr = xr * absatt_ref[0, :]

    for h in range(H):
        xlh_s[h] = xl[:, h * C:(h + 1) * C]
        xrh_s[h] = vr[:, h * C:(h + 1) * C].astype(jnp.bfloat16)
        xlt_s[h] = vl[:, h * C:(h + 1) * C].T.astype(jnp.bfloat16)

    for h in range(H):
        xl_h = xlh_s[h]
        vl_hT = xlt_s[h]
        sgn_col = jnp.reshape(sgn_ref[h, :], (C, 1))
        al_row = jnp.sum(vl_hT.astype(jnp.float32) * sgn_col, axis=0,
                         keepdims=True)
        base = 1.5 * al_row
        sgn8_h = sgn8_ref[h].astype(jnp.bfloat16)

        def body(j, _, xl_h=xl_h, vl_hT=vl_hT, base=base, sgn8_h=sgn8_h, h=h):
            i0 = j * IBLK
            vr_blk = xrh_s[h, pl.ds(i0, IBLK), :]
            v = vr_blk[:, :, None] + vl_hT[None, :, :]
            absv = jnp.abs(v).reshape(IBLK // G, G * C, N)
            parts = [
                jax.lax.dot_general(
                    sgn8_h, absv[g], (((1,), (0,)), ((), ())),
                    preferred_element_type=jnp.float32)
                for g in range(IBLK // G)
            ]
            e = jnp.concatenate(parts, axis=0) + base
            m = jnp.max(e, axis=1, keepdims=True)
            p = jnp.exp(e - m)
            den = jnp.sum(p, axis=1, keepdims=True)
            o = jnp.dot(p, xl_h, preferred_element_type=jnp.float32) / den
            out_ref[pl.ds(i0, IBLK), h * C:(h + 1) * C] = o
            return 0

        jax.lax.fori_loop(0, N // IBLK, body, 0)

    return out_ref[:, :] + bias_ref[0, :]


def _kernel_body(x_ref, Wl1_ref, bl1_ref, Wr1_ref, br1_ref, att1_ref, aa1_ref,
                 sg1_ref, sg81_ref, bias1_ref, g1_ref, be1_ref, Wl2_ref, bl2_ref,
                 Wr2_ref, br2_ref, att2_ref, aa2_ref, sg2_ref, sg82_ref,
                 bias2_ref, g2_ref, be2_ref,
                 out_ref, hbuf, xlh_s, xrh_s, xlt_s):
    xb = x_ref[0]
    h1 = _gat_layer(xb, Wl1_ref, bl1_ref, Wr1_ref, br1_ref, att1_ref, aa1_ref,
                    sg1_ref, sg81_ref, bias1_ref, hbuf, xlh_s, xrh_s, xlt_s)
    h1 = jax.nn.relu(_layernorm(h1, g1_ref[0, :], be1_ref[0, :]))
    o = _gat_layer(h1, Wl2_ref, bl2_ref, Wr2_ref, br2_ref, att2_ref, aa2_ref,
                   sg2_ref, sg82_ref, bias2_ref, hbuf, xlh_s, xrh_s, xlt_s)
    out_ref[0] = _layernorm(o, g2_ref[0, :], be2_ref[0, :])


def kernel(x, embedding, Wl1, bl1, Wr1, br1, att1, bias1, g1, be1,
           Wl2, bl2, Wr2, br2, att2, bias2, g2, be2):
    del embedding
    B = x.shape[0]

    row = lambda v: v.reshape(1, -1)
    whole = lambda shape: pl.BlockSpec(shape, lambda b: (0,) * len(shape))

    in_specs = [
            pl.BlockSpec((1, N, F_IN), lambda b: (b, 0, 0)),
            whole((F_IN, D)), whole((1, D)), whole((F_IN, D)), whole((1, D)),
            whole((H, C)), whole((1, D)), whole((H, C)), whole((H, G, G * C)),
            whole((1, D)), whole((1, D)), whole((1, D)),
            whole((D, D)), whole((1, D)), whole((D, D)), whole((1, D)),
            whole((H, C)), whole((1, D)), whole((H, C)), whole((H, G, G * C)),
            whole((1, D)), whole((1, D)), whole((1, D)),
    ]

    call = pl.pallas_call(
        _kernel_body,
        grid=(B,),
        in_specs=in_specs,
        out_specs=pl.BlockSpec((1, N, D), lambda b: (b, 0, 0)),
        out_shape=jax.ShapeDtypeStruct((B, N, D), jnp.float32),
        scratch_shapes=[pltpu.VMEM((N, D), jnp.float32),
                        pltpu.VMEM((H, N, C), jnp.float32),
                        pltpu.VMEM((H, N, C), jnp.bfloat16),
                        pltpu.VMEM((H, C, N), jnp.bfloat16)],
    )
    blockdiag = jax.vmap(lambda s: jnp.kron(jnp.eye(G, dtype=jnp.float32),
                                            s[None, :]))
    sg1 = jnp.sign(att1)
    sg2 = jnp.sign(att2)
    return call(x, Wl1, row(bl1), Wr1, row(br1), att1, row(0.4 * jnp.abs(att1)),
                sg1, blockdiag(sg1), row(bias1), row(g1), row(be1),
                Wl2, row(bl2), Wr2, row(br2), att2, row(0.4 * jnp.abs(att2)),
                sg2, blockdiag(sg2), row(bias2), row(g2), row(be2))


# speedup vs baseline: 1.3846x; 1.1100x over previous
import functools

import jax
import jax.numpy as jnp
from jax.experimental import pallas as pl
from jax.experimental.pallas import tpu as pltpu

N = 512
H = 4
C = 64
F_IN = 128
D = H * C
IBLK = 128
G = 8
NEG_SLOPE = 0.2


def _layernorm(x, g, b, eps=1e-5):
    m = jnp.mean(x, axis=-1, keepdims=True)
    v = jnp.mean((x - m) ** 2, axis=-1, keepdims=True)
    return (x - m) / jnp.sqrt(v + eps) * g + b


def _gat_layer(xin, Wl_ref, bl_ref, Wr_ref, br_ref, att_ref, absatt_ref,
               sgn_ref, sgn8_ref, bias_ref, out_ref, xlh_s, xrh_s, xlt_s):
    xl = jnp.dot(xin, Wl_ref[:, :], preferred_element_type=jnp.float32) + bl_ref[0, :]
    xr = jnp.dot(xin, Wr_ref[:, :], preferred_element_type=jnp.float32) + br_ref[0, :]
    vl = xl * absatt_ref[0, :]
    vr = xr * absatt_ref[0, :]

    for h in range(H):
        xlh_s[h] = xl[:, h * C:(h + 1) * C]
        xrh_s[h] = vr[:, h * C:(h + 1) * C].astype(jnp.bfloat16)
        xlt_s[h] = vl[:, h * C:(h + 1) * C].T.astype(jnp.bfloat16)

    for h in range(H):
        xl_h = xlh_s[h]
        vl_hT = xlt_s[h]
        sgn_col = jnp.reshape(sgn_ref[h, :], (C, 1))
        al_row = jnp.sum(vl_hT.astype(jnp.float32) * sgn_col, axis=0,
                         keepdims=True)
        base = 1.5 * al_row
        sgn8_h = sgn8_ref[h].astype(jnp.bfloat16)

        def compute_e(i0, vl_hT=vl_hT, base=base, sgn8_h=sgn8_h, h=h):
            vr_blk = xrh_s[h, pl.ds(i0, IBLK), :]
            v = vr_blk[:, :, None] + vl_hT[None, :, :]
            absv = jnp.abs(v).reshape(IBLK // G, G * C, N)
            parts = [
                jax.lax.dot_general(
                    sgn8_h, absv[g], (((1,), (0,)), ((), ())),
                    preferred_element_type=jnp.float32)
                for g in range(IBLK // G)
            ]
            return jnp.concatenate(parts, axis=0) + base

        def soft_agg(e, i0, xl_h=xl_h, h=h):
            m = jnp.max(e, axis=1, keepdims=True)
            p = jnp.exp(e - m)
            den = jnp.sum(p, axis=1, keepdims=True)
            o = jnp.dot(p, xl_h, preferred_element_type=jnp.float32) / den
            out_ref[pl.ds(i0, IBLK), h * C:(h + 1) * C] = o

        def body(j, e_prev):
            e_new = compute_e(j * IBLK)
            soft_agg(e_prev, (j - 1) * IBLK)
            return e_new

        e_last = jax.lax.fori_loop(1, N // IBLK, body, compute_e(0))
        soft_agg(e_last, N - IBLK)

    return out_ref[:, :] + bias_ref[0, :]


def _kernel_body(x_ref, Wl1_ref, bl1_ref, Wr1_ref, br1_ref, att1_ref, aa1_ref,
                 sg1_ref, sg81_ref, bias1_ref, g1_ref, be1_ref, Wl2_ref, bl2_ref,
                 Wr2_ref, br2_ref, att2_ref, aa2_ref, sg2_ref, sg82_ref,
                 bias2_ref, g2_ref, be2_ref,
                 out_ref, hbuf, xlh_s, xrh_s, xlt_s):
    xb = x_ref[0]
    h1 = _gat_layer(xb, Wl1_ref, bl1_ref, Wr1_ref, br1_ref, att1_ref, aa1_ref,
                    sg1_ref, sg81_ref, bias1_ref, hbuf, xlh_s, xrh_s, xlt_s)
    h1 = jax.nn.relu(_layernorm(h1, g1_ref[0, :], be1_ref[0, :]))
    o = _gat_layer(h1, Wl2_ref, bl2_ref, Wr2_ref, br2_ref, att2_ref, aa2_ref,
                   sg2_ref, sg82_ref, bias2_ref, hbuf, xlh_s, xrh_s, xlt_s)
    out_ref[0] = _layernorm(o, g2_ref[0, :], be2_ref[0, :])


def kernel(x, embedding, Wl1, bl1, Wr1, br1, att1, bias1, g1, be1,
           Wl2, bl2, Wr2, br2, att2, bias2, g2, be2):
    del embedding
    B = x.shape[0]

    row = lambda v: v.reshape(1, -1)
    whole = lambda shape: pl.BlockSpec(shape, lambda b: (0,) * len(shape))

    in_specs = [
            pl.BlockSpec((1, N, F_IN), lambda b: (b, 0, 0)),
            whole((F_IN, D)), whole((1, D)), whole((F_IN, D)), whole((1, D)),
            whole((H, C)), whole((1, D)), whole((H, C)), whole((H, G, G * C)),
            whole((1, D)), whole((1, D)), whole((1, D)),
            whole((D, D)), whole((1, D)), whole((D, D)), whole((1, D)),
            whole((H, C)), whole((1, D)), whole((H, C)), whole((H, G, G * C)),
            whole((1, D)), whole((1, D)), whole((1, D)),
    ]

    call = pl.pallas_call(
        _kernel_body,
        grid=(B,),
        in_specs=in_specs,
        out_specs=pl.BlockSpec((1, N, D), lambda b: (b, 0, 0)),
        out_shape=jax.ShapeDtypeStruct((B, N, D), jnp.float32),
        scratch_shapes=[pltpu.VMEM((N, D), jnp.float32),
                        pltpu.VMEM((H, N, C), jnp.float32),
                        pltpu.VMEM((H, N, C), jnp.bfloat16),
                        pltpu.VMEM((H, C, N), jnp.bfloat16)],
    )
    blockdiag = jax.vmap(lambda s: jnp.kron(jnp.eye(G, dtype=jnp.float32),
                                            s[None, :]))
    sg1 = jnp.sign(att1)
    sg2 = jnp.sign(att2)
    return call(x, Wl1, row(bl1), Wr1, row(br1), att1, row(0.4 * jnp.abs(att1)),
                sg1, blockdiag(sg1), row(bias1), row(g1), row(be1),
                Wl2, row(bl2), Wr2, row(br2), att2, row(0.4 * jnp.abs(att2)),
                sg2, blockdiag(sg2), row(bias2), row(g2), row(be2))


# speedup vs baseline: 1.4023x; 1.0128x over previous
import functools

import jax
import jax.numpy as jnp
from jax.experimental import pallas as pl
from jax.experimental.pallas import tpu as pltpu

N = 512
H = 4
C = 64
F_IN = 128
D = H * C
IBLK = 128
G = 8
NEG_SLOPE = 0.2


def _layernorm(x, g, b, eps=1e-5):
    m = jnp.mean(x, axis=-1, keepdims=True)
    v = jnp.mean((x - m) ** 2, axis=-1, keepdims=True)
    return (x - m) / jnp.sqrt(v + eps) * g + b


def _gat_layer(xin, Wl_ref, bl_ref, Wr_ref, br_ref, att_ref, absatt_ref,
               sgn_ref, sgn8_ref, bias_ref, out_ref, xlh_s, xrh_s, xlt_s,
               base_s, outh_s):
    xl = jnp.dot(xin, Wl_ref[:, :], preferred_element_type=jnp.float32) + bl_ref[0, :]
    xr = jnp.dot(xin, Wr_ref[:, :], preferred_element_type=jnp.float32) + br_ref[0, :]
    vl = xl * absatt_ref[0, :]
    vr = xr * absatt_ref[0, :]

    for h in range(H):
        xlh_s[h] = xl[:, h * C:(h + 1) * C]
        xrh_s[h] = vr[:, h * C:(h + 1) * C].astype(jnp.bfloat16)
        xlt_s[h] = vl[:, h * C:(h + 1) * C].T.astype(jnp.bfloat16)

    for h in range(H):
        sgn_col = jnp.reshape(sgn_ref[h, :], (C, 1))
        al_row = jnp.sum(xlt_s[h].astype(jnp.float32) * sgn_col, axis=0,
                         keepdims=True)
        base_s[h, pl.ds(0, 1), :] = 1.5 * al_row

    NB = N // IBLK

    def compute_e(h, j):
        vr_blk = xrh_s[h, pl.ds(j * IBLK, IBLK), :]
        vl_hT = xlt_s[h]
        sgn8_h = sgn8_ref[h].astype(jnp.bfloat16)
        v = vr_blk[:, :, None] + vl_hT[None, :, :]
        absv = jnp.abs(v).reshape(IBLK // G, G * C, N)
        parts = [
            jax.lax.dot_general(
                sgn8_h, absv[g], (((1,), (0,)), ((), ())),
                preferred_element_type=jnp.float32)
            for g in range(IBLK // G)
        ]
        return jnp.concatenate(parts, axis=0) + base_s[h, 0:1, :]

    def soft_agg(e, h, j):
        m = jnp.max(e, axis=1, keepdims=True)
        p = jnp.exp(e - m)
        den = jnp.sum(p, axis=1, keepdims=True)
        o = jnp.dot(p, xlh_s[h], preferred_element_type=jnp.float32) / den
        outh_s[h, pl.ds(j * IBLK, IBLK), :] = o

    def body(s, e_prev):
        e_new = compute_e(s // NB, s % NB)
        sp = s - 1
        soft_agg(e_prev, sp // NB, sp % NB)
        return e_new

    e_last = jax.lax.fori_loop(1, H * NB, body, compute_e(0, 0))
    soft_agg(e_last, H - 1, NB - 1)

    for h in range(H):
        out_ref[:, h * C:(h + 1) * C] = outh_s[h]

    return out_ref[:, :] + bias_ref[0, :]


def _kernel_body(x_ref, Wl1_ref, bl1_ref, Wr1_ref, br1_ref, att1_ref, aa1_ref,
                 sg1_ref, sg81_ref, bias1_ref, g1_ref, be1_ref, Wl2_ref, bl2_ref,
                 Wr2_ref, br2_ref, att2_ref, aa2_ref, sg2_ref, sg82_ref,
                 bias2_ref, g2_ref, be2_ref,
                 out_ref, hbuf, xlh_s, xrh_s, xlt_s, base_s, outh_s):
    xb = x_ref[0]
    h1 = _gat_layer(xb, Wl1_ref, bl1_ref, Wr1_ref, br1_ref, att1_ref, aa1_ref,
                    sg1_ref, sg81_ref, bias1_ref, hbuf, xlh_s, xrh_s, xlt_s,
                    base_s, outh_s)
    h1 = jax.nn.relu(_layernorm(h1, g1_ref[0, :], be1_ref[0, :]))
    o = _gat_layer(h1, Wl2_ref, bl2_ref, Wr2_ref, br2_ref, att2_ref, aa2_ref,
                   sg2_ref, sg82_ref, bias2_ref, hbuf, xlh_s, xrh_s, xlt_s,
                   base_s, outh_s)
    out_ref[0] = _layernorm(o, g2_ref[0, :], be2_ref[0, :])


def kernel(x, embedding, Wl1, bl1, Wr1, br1, att1, bias1, g1, be1,
           Wl2, bl2, Wr2, br2, att2, bias2, g2, be2):
    del embedding
    B = x.shape[0]

    row = lambda v: v.reshape(1, -1)
    whole = lambda shape: pl.BlockSpec(shape, lambda b: (0,) * len(shape))

    in_specs = [
            pl.BlockSpec((1, N, F_IN), lambda b: (b, 0, 0)),
            whole((F_IN, D)), whole((1, D)), whole((F_IN, D)), whole((1, D)),
            whole((H, C)), whole((1, D)), whole((H, C)), whole((H, G, G * C)),
            whole((1, D)), whole((1, D)), whole((1, D)),
            whole((D, D)), whole((1, D)), whole((D, D)), whole((1, D)),
            whole((H, C)), whole((1, D)), whole((H, C)), whole((H, G, G * C)),
            whole((1, D)), whole((1, D)), whole((1, D)),
    ]

    call = pl.pallas_call(
        _kernel_body,
        grid=(B,),
        in_specs=in_specs,
        out_specs=pl.BlockSpec((1, N, D), lambda b: (b, 0, 0)),
        out_shape=jax.ShapeDtypeStruct((B, N, D), jnp.float32),
        scratch_shapes=[pltpu.VMEM((N, D), jnp.float32),
                        pltpu.VMEM((H, N, C), jnp.float32),
                        pltpu.VMEM((H, N, C), jnp.bfloat16),
                        pltpu.VMEM((H, C, N), jnp.bfloat16),
                        pltpu.VMEM((H, 8, N), jnp.float32),
                        pltpu.VMEM((H, N, C), jnp.float32)],
    )
    blockdiag = jax.vmap(lambda s: jnp.kron(jnp.eye(G, dtype=jnp.float32),
                                            s[None, :]))
    sg1 = jnp.sign(att1)
    sg2 = jnp.sign(att2)
    return call(x, Wl1, row(bl1), Wr1, row(br1), att1, row(0.4 * jnp.abs(att1)),
                sg1, blockdiag(sg1), row(bias1), row(g1), row(be1),
                Wl2, row(bl2), Wr2, row(br2), att2, row(0.4 * jnp.abs(att2)),
                sg2, blockdiag(sg2), row(bias2), row(g2), row(be2))
